# Initial kernel scaffold; baseline (speedup 1.0000x reference)
import jax
import jax.numpy as jnp
from jax.experimental import pallas as pl


def kernel(x, emb, pos_emb):
    raise NotImplementedError("write your pallas kernel here")


# speedup vs baseline: 3.4472x; 3.4472x over previous
import functools

import jax
import jax.numpy as jnp
from jax import lax
from jax.experimental import pallas as pl
from jax.experimental.pallas import tpu as pltpu
from jax.experimental.pallas import tpu_sc as plsc

N_VOCAB = 100000
D_MODEL = 128
BATCH = 4
SEQ = 8192

NUM_CORES = 2
NUM_SUBCORES = 16
NUM_WORKERS = NUM_CORES * NUM_SUBCORES
S_PER_W = SEQ // NUM_WORKERS
SUB = 128
K = (BATCH * S_PER_W) // SUB
LANES = 16

_mesh = plsc.VectorSubcoreMesh(core_axis_name="c", subcore_axis_name="s")


@functools.partial(
    pl.kernel,
    mesh=_mesh,
    out_type=jax.ShapeDtypeStruct((BATCH * SEQ, D_MODEL), jnp.float32),
    scratch_types=[
        pltpu.VMEM((K, SUB), jnp.int32),
        pltpu.VMEM((S_PER_W, D_MODEL), jnp.float32),
        pltpu.VMEM((SUB, D_MODEL), jnp.float32),
        pltpu.VMEM((SUB, D_MODEL), jnp.float32),
        pltpu.SemaphoreType.DMA,
        pltpu.SemaphoreType.DMA,
    ],
)
def _emb_kernel(x_hbm, emb_hbm, pos_hbm, out_hbm,
                idx_v, pos_v, rows_a, rows_b, sem_a, sem_b):
    wid = lax.axis_index("s") * NUM_CORES + lax.axis_index("c")
    s0 = wid * S_PER_W

    pltpu.sync_copy(pos_hbm.at[pl.ds(s0, S_PER_W)], pos_v)

    for k in range(K):
        off = (k // 2) * SEQ + s0 + (k % 2) * SUB
        pltpu.sync_copy(x_hbm.at[pl.ds(off, SUB)], idx_v.at[k])

    bufs = (rows_a, rows_b)
    sems = (sem_a, sem_b)
    copies = [None, None]
    copies[0] = pltpu.async_copy(emb_hbm.at[idx_v.at[0]], bufs[0], sems[0])

    for k in range(K):
        cur = bufs[k % 2]
        copies[k % 2].wait()
        if k + 1 < K:
            nxt = (k + 1) % 2
            copies[nxt] = pltpu.async_copy(
                emb_hbm.at[idx_v.at[k + 1]], bufs[nxt], sems[nxt])

        jb = (k % 2) * SUB

        def add_pos(r, carry, cur=cur, jb=jb):
            for cc in range(D_MODEL // LANES):
                c = cc * LANES
                cur[r, pl.ds(c, LANES)] = (
                    cur[r, pl.ds(c, LANES)] + pos_v[jb + r, pl.ds(c, LANES)])
            return carry

        lax.fori_loop(0, SUB, add_pos, 0)

        out_row = (k // 2) * SEQ + s0 + (k % 2) * SUB
        pltpu.sync_copy(cur, out_hbm.at[pl.ds(out_row, SUB)])


def kernel(x, emb, pos_emb):
    xf = x.reshape(-1).astype(jnp.int32)
    out = _emb_kernel(xf, emb, pos_emb)
    return out.reshape(x.shape[0], x.shape[1], D_MODEL)


# speedup vs baseline: 3.5459x; 1.0286x over previous
import functools

import jax
import jax.numpy as jnp
from jax import lax
from jax.experimental import pallas as pl
from jax.experimental.pallas import tpu as pltpu
from jax.experimental.pallas import tpu_sc as plsc

N_VOCAB = 100000
D_MODEL = 128
BATCH = 4
SEQ = 8192

NUM_CORES = 2
NUM_SUBCORES = 16
NUM_WORKERS = NUM_CORES * NUM_SUBCORES
S_PER_W = SEQ // NUM_WORKERS
SUB = 128
K = (BATCH * S_PER_W) // SUB
LANES = 16

_mesh = plsc.VectorSubcoreMesh(core_axis_name="c", subcore_axis_name="s")


NBUF = 4


@functools.partial(
    pl.kernel,
    mesh=_mesh,
    out_type=jax.ShapeDtypeStruct((BATCH * SEQ, D_MODEL), jnp.float32),
    scratch_types=[
        pltpu.VMEM((K, SUB), jnp.int32),
        pltpu.VMEM((S_PER_W, D_MODEL), jnp.float32),
    ] + [pltpu.VMEM((SUB, D_MODEL), jnp.float32) for _ in range(NBUF)]
      + [pltpu.SemaphoreType.DMA for _ in range(2 * NBUF + 1)],
)
def _emb_kernel(x_hbm, emb_hbm, pos_hbm, out_hbm, idx_v, pos_v, *rest):
    bufs = rest[:NBUF]
    gsems = rest[NBUF:2 * NBUF]
    osems = rest[2 * NBUF:3 * NBUF]
    psem = rest[3 * NBUF]

    wid = lax.axis_index("s") * NUM_CORES + lax.axis_index("c")
    s0 = wid * S_PER_W

    pos_copy = pltpu.async_copy(pos_hbm.at[pl.ds(s0, S_PER_W)], pos_v, psem)

    for k in range(K):
        off = (k // 2) * SEQ + s0 + (k % 2) * SUB
        pltpu.sync_copy(x_hbm.at[pl.ds(off, SUB)], idx_v.at[k])

    gathers = [None] * NBUF
    outs = [None] * NBUF
    for k in range(NBUF - 1):
        gathers[k] = pltpu.async_copy(
            emb_hbm.at[idx_v.at[k]], bufs[k], gsems[k])

    pos_copy.wait()

    for k in range(K):
        b = k % NBUF
        if k + NBUF - 1 < K:
            nb = (k + NBUF - 1) % NBUF
            if outs[nb] is not None:
                outs[nb].wait()
            gathers[nb] = pltpu.async_copy(
                emb_hbm.at[idx_v.at[k + NBUF - 1]], bufs[nb], gsems[nb])

        cur = bufs[b]
        gathers[b].wait()

        jb = (k % 2) * SUB

        def add_pos(r, carry, cur=cur, jb=jb):
            for cc in range(D_MODEL // LANES):
                c = cc * LANES
                cur[r, pl.ds(c, LANES)] = (
                    cur[r, pl.ds(c, LANES)] + pos_v[jb + r, pl.ds(c, LANES)])
            return carry

        lax.fori_loop(0, SUB, add_pos, 0)

        out_row = (k // 2) * SEQ + s0 + (k % 2) * SUB
        outs[b] = pltpu.async_copy(cur, out_hbm.at[pl.ds(out_row, SUB)], osems[b])

    for b in range(NBUF):
        if outs[b] is not None:
            outs[b].wait()


def kernel(x, emb, pos_emb):
    xf = x.reshape(-1).astype(jnp.int32)
    out = _emb_kernel(xf, emb, pos_emb)
    return out.reshape(x.shape[0], x.shape[1], D_MODEL)


# speedup vs baseline: 3.8840x; 1.0953x over previous
import functools

import jax
import jax.numpy as jnp
from jax import lax
from jax.experimental import pallas as pl
from jax.experimental.pallas import tpu as pltpu
from jax.experimental.pallas import tpu_sc as plsc

N_VOCAB = 100000
D_MODEL = 128
BATCH = 4
SEQ = 8192

NUM_CORES = 2
NUM_SUBCORES = 16
NUM_WORKERS = NUM_CORES * NUM_SUBCORES
S_PER_W = SEQ // NUM_WORKERS
SUB = 128
K = (BATCH * S_PER_W) // SUB
LANES = 16
NBUF = 4

_mesh = plsc.VectorSubcoreMesh(core_axis_name="c", subcore_axis_name="s")


@functools.partial(
    pl.kernel,
    mesh=_mesh,
    out_type=jax.ShapeDtypeStruct((BATCH, SEQ, D_MODEL), jnp.float32),
    scratch_types=[
        pltpu.VMEM((K, SUB), jnp.int32),
        pltpu.VMEM((S_PER_W, D_MODEL), jnp.float32),
    ] + [pltpu.VMEM((SUB, D_MODEL), jnp.float32) for _ in range(NBUF)]
      + [pltpu.SemaphoreType.DMA for _ in range(2 * NBUF + 2)],
)
def _emb_kernel(x_hbm, emb_hbm, pos_hbm, out_hbm, idx_v, pos_v, *rest):
    bufs = rest[:NBUF]
    gsems = rest[NBUF:2 * NBUF]
    osems = rest[2 * NBUF:3 * NBUF]
    psem = rest[3 * NBUF]
    isem = rest[3 * NBUF + 1]

    wid = lax.axis_index("s") * NUM_CORES + lax.axis_index("c")
    s0 = wid * S_PER_W

    pos_copy = pltpu.async_copy(pos_hbm.at[pl.ds(s0, S_PER_W)], pos_v, psem)

    idx_copies = []
    for k in range(K):
        idx_copies.append(pltpu.async_copy(
            x_hbm.at[k // 2, pl.ds(s0 + (k % 2) * SUB, SUB)], idx_v.at[k], isem))
    for c in idx_copies:
        c.wait()

    gathers = [None] * NBUF
    outs = [None] * NBUF
    for k in range(NBUF - 1):
        gathers[k] = pltpu.async_copy(
            emb_hbm.at[idx_v.at[k]], bufs[k], gsems[k])

    pos_copy.wait()

    for k in range(K):
        b = k % NBUF
        if k + NBUF - 1 < K:
            nb = (k + NBUF - 1) % NBUF
            if outs[nb] is not None:
                outs[nb].wait()
            gathers[nb] = pltpu.async_copy(
                emb_hbm.at[idx_v.at[k + NBUF - 1]], bufs[nb], gsems[nb])

        cur = bufs[b]
        gathers[b].wait()

        jb = (k % 2) * SUB

        def add_pos(r, carry, cur=cur, jb=jb):
            for cc in range(D_MODEL // LANES):
                c = cc * LANES
                cur[r, pl.ds(c, LANES)] = (
                    cur[r, pl.ds(c, LANES)] + pos_v[jb + r, pl.ds(c, LANES)])
            return carry

        lax.fori_loop(0, SUB, add_pos, 0)

        outs[b] = pltpu.async_copy(
            cur, out_hbm.at[k // 2, pl.ds(s0 + (k % 2) * SUB, SUB)], osems[b])

    for b in range(NBUF):
        if outs[b] is not None:
            outs[b].wait()


def kernel(x, emb, pos_emb):
    return _emb_kernel(x.astype(jnp.int32), emb, pos_emb)
